# baseline (speedup 1.0000x reference)
import functools

import jax
import jax.numpy as jnp
from jax import lax
from jax.experimental import pallas as pl
from jax.experimental.pallas import tpu as pltpu
from jax.experimental.pallas import tpu_sc as plsc

D = 128
LANES = 16
CHUNK = 128


def _sc_body(num_cores):
    def body(ent_idx_hbm, type_idx_hbm, ent_hbm, type_hbm, out_hbm,
             idx_e, idx_t, erows, trows, outv, sem_e, sem_t):
        wid = lax.axis_index("s") * num_cores + lax.axis_index("c")
        nch = idx_e.shape[0]
        pltpu.sync_copy(ent_idx_hbm.at[wid], idx_e)
        pltpu.sync_copy(type_idx_hbm.at[wid], idx_t)
        for j in range(nch):
            ce = pltpu.async_copy(ent_hbm.at[idx_e.at[j]], erows, sem_e)
            ct = pltpu.async_copy(type_hbm.at[idx_t.at[j]], trows, sem_t)
            ce.wait()
            ct.wait()
            for g in range(CHUNK // LANES):
                rows = jnp.arange(LANES, dtype=jnp.int32) + (g * LANES)

                def col(c, acc, rows=rows):
                    cc = jnp.full((LANES,), c, jnp.int32)
                    e = plsc.load_gather(erows, [rows, cc])
                    t = plsc.load_gather(trows, [rows, cc])
                    return acc + e * t

                acc = lax.fori_loop(0, D, col, jnp.zeros((LANES,), jnp.float32))
                outv[pl.ds(j * CHUNK + g * LANES, LANES)] = acc
        pltpu.sync_copy(outv, out_hbm.at[wid])

    return body


def kernel(entity, pos_type, ent_emb, type_embedding):
    B = entity.shape[0]
    info = plsc.get_sparse_core_info()
    nw = info.num_cores * info.num_subcores
    bpw = B // nw
    nch = bpw // CHUNK
    mesh = plsc.VectorSubcoreMesh(core_axis_name="c", subcore_axis_name="s")
    ent_idx = entity.astype(jnp.int32).reshape(nw, nch, CHUNK)
    type_idx = pos_type.astype(jnp.int32).reshape(nw, nch, CHUNK)
    k = functools.partial(
        pl.kernel,
        mesh=mesh,
        compiler_params=pltpu.CompilerParams(needs_layout_passes=False),
        out_type=jax.ShapeDtypeStruct((nw, bpw), jnp.float32),
        scratch_types=[
            pltpu.VMEM((nch, CHUNK), jnp.int32),
            pltpu.VMEM((nch, CHUNK), jnp.int32),
            pltpu.VMEM((CHUNK, D), jnp.float32),
            pltpu.VMEM((CHUNK, D), jnp.float32),
            pltpu.VMEM((bpw,), jnp.float32),
            pltpu.SemaphoreType.DMA,
            pltpu.SemaphoreType.DMA,
        ],
    )(_sc_body(info.num_cores))
    out = k(ent_idx, type_idx, ent_emb, type_embedding)
    return out.reshape(B, 1)


# speedup vs baseline: 1.8243x; 1.8243x over previous
import functools

import jax
import jax.numpy as jnp
from jax import lax
from jax.experimental import pallas as pl
from jax.experimental.pallas import tpu as pltpu
from jax.experimental.pallas import tpu_sc as plsc

D = 128
LANES = 16
CHUNK = 128
NBLK = D // LANES

_GDN = lax.GatherDimensionNumbers(
    offset_dims=(), collapsed_slice_dims=(0,), start_index_map=(0,))


def _lane_shuffle(x, idx):
    return lax.gather(x, idx[:, None], _GDN, (1,),
                      mode=lax.GatherScatterMode.PROMISE_IN_BOUNDS)


def _row_dot(erows, trows, row):
    prods = []
    for b in range(NBLK):
        e = erows[row, pl.ds(b * LANES, LANES)]
        t = trows[row, pl.ds(b * LANES, LANES)]
        prods.append(e * t)
    while len(prods) > 1:
        prods = [prods[i] + prods[i + 1] for i in range(0, len(prods), 2)]
    x = prods[0]
    iota = jnp.arange(LANES, dtype=jnp.int32)
    for s in (8, 4, 2, 1):
        x = x + _lane_shuffle(x, iota ^ s)
    return x


def _sc_body(num_cores):
    def body(ent_idx_hbm, type_idx_hbm, ent_hbm, type_hbm, out_hbm,
             idx_e, idx_t, erows, trows, outv,
             sem_e0, sem_e1, sem_t0, sem_t1):
        wid = lax.axis_index("s") * num_cores + lax.axis_index("c")
        nch = idx_e.shape[0]
        sem_e = (sem_e0, sem_e1)
        sem_t = (sem_t0, sem_t1)
        pltpu.sync_copy(ent_idx_hbm.at[wid], idx_e)
        pltpu.sync_copy(type_idx_hbm.at[wid], idx_t)

        def start(j):
            p = j % 2
            he = pltpu.async_copy(ent_hbm.at[idx_e.at[j]], erows.at[p], sem_e[p])
            ht = pltpu.async_copy(type_hbm.at[idx_t.at[j]], trows.at[p], sem_t[p])
            return he, ht

        iota = jnp.arange(LANES, dtype=jnp.int32)
        masks = [iota == r for r in range(LANES)]

        handles = {}
        for j in range(min(2, nch)):
            handles[j] = start(j)
        for j in range(nch):
            p = j % 2
            he, ht = handles.pop(j)
            he.wait()
            ht.wait()
            ebuf = erows.at[p]
            tbuf = trows.at[p]

            def group_body(g, _, ebuf=ebuf, tbuf=tbuf, j=j):
                res = jnp.zeros((LANES,), jnp.float32)
                base = g * LANES
                for r in range(LANES):
                    x = _row_dot(ebuf, tbuf, base + r)
                    res = jnp.where(masks[r], x, res)
                outv[pl.ds(j * CHUNK + base, LANES)] = res
                return 0

            lax.fori_loop(0, CHUNK // LANES, group_body, 0)
            if j + 2 < nch:
                handles[j + 2] = start(j + 2)
        pltpu.sync_copy(outv, out_hbm.at[wid])

    return body


def kernel(entity, pos_type, ent_emb, type_embedding):
    B = entity.shape[0]
    info = plsc.get_sparse_core_info()
    nw = info.num_cores * info.num_subcores
    bpw = B // nw
    nch = bpw // CHUNK
    mesh = plsc.VectorSubcoreMesh(core_axis_name="c", subcore_axis_name="s")
    ent_idx = entity.astype(jnp.int32).reshape(nw, nch, CHUNK)
    type_idx = pos_type.astype(jnp.int32).reshape(nw, nch, CHUNK)
    k = functools.partial(
        pl.kernel,
        mesh=mesh,
        compiler_params=pltpu.CompilerParams(needs_layout_passes=False),
        out_type=jax.ShapeDtypeStruct((nw, bpw), jnp.float32),
        scratch_types=[
            pltpu.VMEM((nch, CHUNK), jnp.int32),
            pltpu.VMEM((nch, CHUNK), jnp.int32),
            pltpu.VMEM((2, CHUNK, D), jnp.float32),
            pltpu.VMEM((2, CHUNK, D), jnp.float32),
            pltpu.VMEM((bpw,), jnp.float32),
            pltpu.SemaphoreType.DMA,
            pltpu.SemaphoreType.DMA,
            pltpu.SemaphoreType.DMA,
            pltpu.SemaphoreType.DMA,
        ],
    )(_sc_body(info.num_cores))
    out = k(ent_idx, type_idx, ent_emb, type_embedding)
    return out.reshape(B, 1)


# speedup vs baseline: 2.2608x; 1.2393x over previous
import functools

import jax
import jax.numpy as jnp
from jax import lax
from jax.experimental import pallas as pl
from jax.experimental.pallas import tpu as pltpu
from jax.experimental.pallas import tpu_sc as plsc

D = 128
LANES = 16
CHUNK = 128
NBLK = D // LANES

_GDN = lax.GatherDimensionNumbers(
    offset_dims=(), collapsed_slice_dims=(0,), start_index_map=(0,))


def _lane_shuffle(x, idx):
    return lax.gather(x, idx[:, None], _GDN, (1,),
                      mode=lax.GatherScatterMode.PROMISE_IN_BOUNDS)


def _row_partial(erows, trows, row):
    prods = []
    for b in range(NBLK):
        e = erows[row, pl.ds(b * LANES, LANES)]
        t = trows[row, pl.ds(b * LANES, LANES)]
        prods.append(e * t)
    while len(prods) > 1:
        prods = [prods[i] + prods[i + 1] for i in range(0, len(prods), 2)]
    return prods[0]


def _tree_reduce(vs):
    iota = jnp.arange(LANES, dtype=jnp.int32)
    s = 1
    while len(vs) > 1:
        mask = (iota & s) == 0
        perm = iota ^ s
        nxt = []
        for i in range(0, len(vs), 2):
            a, b = vs[i], vs[i + 1]
            keep = jnp.where(mask, a, b)
            other = _lane_shuffle(jnp.where(mask, b, a), perm)
            nxt.append(keep + other)
        vs = nxt
        s *= 2
    return vs[0]


def _sc_body(num_cores):
    def body(ent_idx_hbm, type_idx_hbm, ent_hbm, type_hbm, out_hbm,
             idx_e, idx_t, erows, trows, outv,
             sem_e0, sem_e1, sem_t0, sem_t1):
        wid = lax.axis_index("s") * num_cores + lax.axis_index("c")
        nch = idx_e.shape[0]
        sem_e = (sem_e0, sem_e1)
        sem_t = (sem_t0, sem_t1)
        pltpu.sync_copy(ent_idx_hbm.at[wid], idx_e)
        pltpu.sync_copy(type_idx_hbm.at[wid], idx_t)

        def start(j):
            p = j % 2
            he = pltpu.async_copy(ent_hbm.at[idx_e.at[j]], erows.at[p], sem_e[p])
            ht = pltpu.async_copy(type_hbm.at[idx_t.at[j]], trows.at[p], sem_t[p])
            return he, ht

        handles = {}
        for j in range(min(2, nch)):
            handles[j] = start(j)
        for j in range(nch):
            p = j % 2
            he, ht = handles.pop(j)
            he.wait()
            ht.wait()
            ebuf = erows.at[p]
            tbuf = trows.at[p]

            def group_body(g, _, ebuf=ebuf, tbuf=tbuf, j=j):
                base = g * LANES
                vs = [_row_partial(ebuf, tbuf, base + r) for r in range(LANES)]
                outv[pl.ds(j * CHUNK + base, LANES)] = _tree_reduce(vs)
                return 0

            lax.fori_loop(0, CHUNK // LANES, group_body, 0)
            if j + 2 < nch:
                handles[j + 2] = start(j + 2)
        pltpu.sync_copy(outv, out_hbm.at[wid])

    return body


def kernel(entity, pos_type, ent_emb, type_embedding):
    B = entity.shape[0]
    info = plsc.get_sparse_core_info()
    nw = info.num_cores * info.num_subcores
    bpw = B // nw
    nch = bpw // CHUNK
    mesh = plsc.VectorSubcoreMesh(core_axis_name="c", subcore_axis_name="s")
    ent_idx = entity.astype(jnp.int32).reshape(nw, nch, CHUNK)
    type_idx = pos_type.astype(jnp.int32).reshape(nw, nch, CHUNK)
    k = functools.partial(
        pl.kernel,
        mesh=mesh,
        compiler_params=pltpu.CompilerParams(needs_layout_passes=False),
        out_type=jax.ShapeDtypeStruct((nw, bpw), jnp.float32),
        scratch_types=[
            pltpu.VMEM((nch, CHUNK), jnp.int32),
            pltpu.VMEM((nch, CHUNK), jnp.int32),
            pltpu.VMEM((2, CHUNK, D), jnp.float32),
            pltpu.VMEM((2, CHUNK, D), jnp.float32),
            pltpu.VMEM((bpw,), jnp.float32),
            pltpu.SemaphoreType.DMA,
            pltpu.SemaphoreType.DMA,
            pltpu.SemaphoreType.DMA,
            pltpu.SemaphoreType.DMA,
        ],
    )(_sc_body(info.num_cores))
    out = k(ent_idx, type_idx, ent_emb, type_embedding)
    return out.reshape(B, 1)
